# baseline (speedup 1.0000x reference)
import functools

import jax
import jax.numpy as jnp
from jax import lax
from jax.experimental import pallas as pl
from jax.experimental.pallas import tpu as pltpu
from jax.experimental.pallas import tpu_sc as plsc

_G = 128
_K = 8


@functools.partial(jax.jit, static_argnums=(2, 3))
def _gather_sc(idx2d, table, B, D):
    info = plsc.get_sparse_core_info()
    NC, NS = info.num_cores, info.num_subcores
    NW = NC * NS
    per_w = B // NW
    S = _K * _G
    n_chunks = per_w // S
    rows_per_w = per_w // _G

    mesh = plsc.VectorSubcoreMesh(core_axis_name="c", subcore_axis_name="s")

    @functools.partial(
        pl.kernel,
        mesh=mesh,
        out_type=jax.ShapeDtypeStruct((B, D), jnp.float32),
        scratch_types=[
            pltpu.VMEM((_K, _G), jnp.int32),
            pltpu.VMEM((S, D), jnp.float32),
            pltpu.SemaphoreType.DMA,
        ],
        compiler_params=pltpu.CompilerParams(use_tc_tiling_on_sc=False),
    )
    def _k(idx_hbm, table_hbm, out_hbm, idx_v, rows_v, sem):
        wid = lax.axis_index("s") * NC + lax.axis_index("c")
        row0 = wid * rows_per_w

        def body(c, carry):
            rbase = row0 + c * _K
            pltpu.sync_copy(idx_hbm.at[pl.ds(rbase, _K)], idx_v)
            copies = []
            for j in range(_K):
                copies.append(
                    pltpu.async_copy(
                        table_hbm.at[idx_v.at[j]],
                        rows_v.at[pl.ds(j * _G, _G)],
                        sem,
                    )
                )
            for cp in copies:
                cp.wait()
            pltpu.sync_copy(rows_v, out_hbm.at[pl.ds(rbase * _G, S)])
            return carry

        lax.fori_loop(0, n_chunks, body, 0)

    return _k(idx2d, table)


def kernel(x, table):
    B = x.shape[0] * x.shape[1]
    D = table.shape[1]
    idx2d = x.reshape(B // _G, _G).astype(jnp.int32)
    out = _gather_sc(idx2d, table, B, D)
    return out.reshape(x.shape[0], x.shape[1], D)
